# Initial kernel scaffold; baseline (speedup 1.0000x reference)
import jax
import jax.numpy as jnp
from jax.experimental import pallas as pl


def kernel(indices, weight):
    raise NotImplementedError("write your pallas kernel here")


# speedup vs baseline: 1.5653x; 1.5653x over previous
import functools

import jax
import jax.numpy as jnp
from jax import lax
from jax.experimental import pallas as pl
from jax.experimental.pallas import tpu as pltpu
from jax.experimental.pallas import tpu_sc as plsc

_VOCAB = 100000
_DIM = 1024
_BATCH = 4
_SEQ = 4096
_NTOK = _BATCH * _SEQ

_NC = 2
_NS = 16
_NW = _NC * _NS
_TOK_PER_W = _NTOK // _NW
_CHUNK = 32
_NCHUNK = _TOK_PER_W // _CHUNK


def _emb_body(idx_hbm, table_hbm, out_hbm, idx_v, rows_v,
              gsem0, gsem1, ssem0, ssem1):
    wid = lax.axis_index("s") * _NC + lax.axis_index("c")
    base = wid * _TOK_PER_W
    pltpu.sync_copy(idx_hbm.at[pl.ds(base, _TOK_PER_W)], idx_v)

    gsems = (gsem0, gsem1)
    ssems = (ssem0, ssem1)

    def gather(i, b):
        return pltpu.async_copy(
            table_hbm.at[idx_v.at[pl.ds(i * _CHUNK, _CHUNK)]],
            rows_v.at[b], gsems[b])

    def store(i, b):
        return pltpu.async_copy(
            rows_v.at[b], out_hbm.at[pl.ds(base + i * _CHUNK, _CHUNK)],
            ssems[b])

    gh = [None, None]
    sh = [None, None]
    gh[0] = gather(0, 0)
    for i in range(_NCHUNK):
        b = i % 2
        nb = 1 - b
        gh[b].wait()
        if i + 1 < _NCHUNK:
            if sh[nb] is not None:
                sh[nb].wait()
            gh[nb] = gather(i + 1, nb)
        sh[b] = store(i, b)
    sh[0].wait()
    sh[1].wait()


@functools.partial(jax.jit, static_argnames=())
def _emb(idx_flat, weight):
    mesh = plsc.VectorSubcoreMesh(core_axis_name="c", subcore_axis_name="s")
    kern = pl.kernel(
        _emb_body,
        out_type=jax.ShapeDtypeStruct((_NTOK, _DIM), jnp.float32),
        mesh=mesh,
        scratch_types=[
            pltpu.VMEM((_TOK_PER_W,), jnp.int32),
            pltpu.VMEM((2, _CHUNK, _DIM), jnp.float32),
            pltpu.SemaphoreType.DMA,
            pltpu.SemaphoreType.DMA,
            pltpu.SemaphoreType.DMA,
            pltpu.SemaphoreType.DMA,
        ],
    )
    return kern(idx_flat, weight)


def kernel(indices, weight):
    idx_flat = indices.reshape(_NTOK).astype(jnp.int32)
    out = _emb(idx_flat, weight)
    return out.reshape(_BATCH, _SEQ, _DIM)


# speedup vs baseline: 1.6543x; 1.0569x over previous
import functools

import jax
import jax.numpy as jnp
from jax import lax
from jax.experimental import pallas as pl
from jax.experimental.pallas import tpu as pltpu
from jax.experimental.pallas import tpu_sc as plsc

_VOCAB = 100000
_DIM = 1024
_BATCH = 4
_SEQ = 4096
_NTOK = _BATCH * _SEQ

_NC = 2
_NS = 16
_NW = _NC * _NS
_TOK_PER_W = _NTOK // _NW
_CHUNK = 32
_NCHUNK = _TOK_PER_W // _CHUNK


_NBUF = 3


def _emb_body(idx_hbm, table_hbm, out_hbm, idx_v, rows_v,
              gsem0, gsem1, gsem2, ssem0, ssem1, ssem2):
    wid = lax.axis_index("s") * _NC + lax.axis_index("c")
    base = wid * _TOK_PER_W
    pltpu.sync_copy(idx_hbm.at[pl.ds(base, _TOK_PER_W)], idx_v)

    gsems = (gsem0, gsem1, gsem2)
    ssems = (ssem0, ssem1, ssem2)

    def gather(i, b):
        return pltpu.async_copy(
            table_hbm.at[idx_v.at[pl.ds(i * _CHUNK, _CHUNK)]],
            rows_v.at[b], gsems[b])

    def store(i, b):
        return pltpu.async_copy(
            rows_v.at[b], out_hbm.at[pl.ds(base + i * _CHUNK, _CHUNK)],
            ssems[b])

    gh = [None] * _NBUF
    sh = [None] * _NBUF
    for b in range(_NBUF - 1):
        gh[b] = gather(b, b)
    for i in range(_NCHUNK):
        b = i % _NBUF
        nb = (i + _NBUF - 1) % _NBUF
        gh[b].wait()
        if i + _NBUF - 1 < _NCHUNK:
            if sh[nb] is not None:
                sh[nb].wait()
            gh[nb] = gather(i + _NBUF - 1, nb)
        sh[b] = store(i, b)
    for b in range(_NBUF):
        if sh[b] is not None:
            sh[b].wait()


@functools.partial(jax.jit, static_argnames=())
def _emb(idx_flat, weight):
    mesh = plsc.VectorSubcoreMesh(core_axis_name="c", subcore_axis_name="s")
    kern = pl.kernel(
        _emb_body,
        out_type=jax.ShapeDtypeStruct((_NTOK, _DIM), jnp.float32),
        mesh=mesh,
        scratch_types=[
            pltpu.VMEM((_TOK_PER_W,), jnp.int32),
            pltpu.VMEM((_NBUF, _CHUNK, _DIM), jnp.float32),
            pltpu.SemaphoreType.DMA,
            pltpu.SemaphoreType.DMA,
            pltpu.SemaphoreType.DMA,
            pltpu.SemaphoreType.DMA,
            pltpu.SemaphoreType.DMA,
            pltpu.SemaphoreType.DMA,
        ],
    )
    return kern(idx_flat, weight)


def kernel(indices, weight):
    idx_flat = indices.reshape(_NTOK).astype(jnp.int32)
    out = _emb(idx_flat, weight)
    return out.reshape(_BATCH, _SEQ, _DIM)


# speedup vs baseline: 1.6719x; 1.0106x over previous
import functools

import jax
import jax.numpy as jnp
from jax import lax
from jax.experimental import pallas as pl
from jax.experimental.pallas import tpu as pltpu
from jax.experimental.pallas import tpu_sc as plsc

_VOCAB = 100000
_DIM = 1024
_BATCH = 4
_SEQ = 4096
_NTOK = _BATCH * _SEQ

_NC = 2
_NS = 16
_NW = _NC * _NS
_TOK_PER_W = _NTOK // _NW
_CHUNK = 16
_NCHUNK = _TOK_PER_W // _CHUNK


_NBUF = 6


def _emb_body(idx_hbm, table_hbm, out_hbm, idx_v, rows_v,
              gsem0, gsem1, gsem2, gsem3, gsem4, gsem5,
              ssem0, ssem1, ssem2, ssem3, ssem4, ssem5):
    wid = lax.axis_index("s") * _NC + lax.axis_index("c")
    base = wid * _TOK_PER_W
    pltpu.sync_copy(idx_hbm.at[pl.ds(base, _TOK_PER_W)], idx_v)

    gsems = (gsem0, gsem1, gsem2, gsem3, gsem4, gsem5)
    ssems = (ssem0, ssem1, ssem2, ssem3, ssem4, ssem5)

    def gather(i, b):
        return pltpu.async_copy(
            table_hbm.at[idx_v.at[pl.ds(i * _CHUNK, _CHUNK)]],
            rows_v.at[b], gsems[b])

    def store(i, b):
        return pltpu.async_copy(
            rows_v.at[b], out_hbm.at[pl.ds(base + i * _CHUNK, _CHUNK)],
            ssems[b])

    gh = [None] * _NBUF
    sh = [None] * _NBUF
    for b in range(_NBUF - 1):
        gh[b] = gather(b, b)
    for i in range(_NCHUNK):
        b = i % _NBUF
        nb = (i + _NBUF - 1) % _NBUF
        gh[b].wait()
        if i + _NBUF - 1 < _NCHUNK:
            if sh[nb] is not None:
                sh[nb].wait()
            gh[nb] = gather(i + _NBUF - 1, nb)
        sh[b] = store(i, b)
    for b in range(_NBUF):
        if sh[b] is not None:
            sh[b].wait()


@functools.partial(jax.jit, static_argnames=())
def _emb(idx_flat, weight):
    mesh = plsc.VectorSubcoreMesh(core_axis_name="c", subcore_axis_name="s")
    kern = pl.kernel(
        _emb_body,
        out_type=jax.ShapeDtypeStruct((_NTOK, _DIM), jnp.float32),
        mesh=mesh,
        scratch_types=[
            pltpu.VMEM((_TOK_PER_W,), jnp.int32),
            pltpu.VMEM((_NBUF, _CHUNK, _DIM), jnp.float32),
            pltpu.SemaphoreType.DMA,
            pltpu.SemaphoreType.DMA,
            pltpu.SemaphoreType.DMA,
            pltpu.SemaphoreType.DMA,
            pltpu.SemaphoreType.DMA,
            pltpu.SemaphoreType.DMA,
            pltpu.SemaphoreType.DMA,
            pltpu.SemaphoreType.DMA,
            pltpu.SemaphoreType.DMA,
            pltpu.SemaphoreType.DMA,
            pltpu.SemaphoreType.DMA,
            pltpu.SemaphoreType.DMA,
        ],
    )
    return kern(idx_flat, weight)


def kernel(indices, weight):
    idx_flat = indices.reshape(_NTOK).astype(jnp.int32)
    out = _emb(idx_flat, weight)
    return out.reshape(_BATCH, _SEQ, _DIM)


# speedup vs baseline: 1.6795x; 1.0045x over previous
import functools

import jax
import jax.numpy as jnp
from jax import lax
from jax.experimental import pallas as pl
from jax.experimental.pallas import tpu as pltpu
from jax.experimental.pallas import tpu_sc as plsc

_VOCAB = 100000
_DIM = 1024
_BATCH = 4
_SEQ = 4096
_NTOK = _BATCH * _SEQ

_NC = 2
_NS = 16
_NW = _NC * _NS
_TOK_PER_W = _NTOK // _NW
_CHUNK = 16
_NCHUNK = _TOK_PER_W // _CHUNK
_NBUF = 4
_NGROUP = _NCHUNK // _NBUF


def _emb_body(idx_hbm, table_hbm, out_hbm, idx_v, rows_v,
              gsem0, gsem1, gsem2, gsem3, ssem0, ssem1, ssem2, ssem3):
    wid = lax.axis_index("s") * _NC + lax.axis_index("c")
    base = wid * _TOK_PER_W
    pltpu.sync_copy(idx_hbm.at[pl.ds(base, _TOK_PER_W)], idx_v)

    gsems = (gsem0, gsem1, gsem2, gsem3)
    ssems = (ssem0, ssem1, ssem2, ssem3)

    def gdesc(j, b):
        return pltpu.make_async_copy(
            table_hbm.at[idx_v.at[pl.ds(j * _CHUNK, _CHUNK)]],
            rows_v.at[b], gsems[b])

    def sdesc(j, b):
        return pltpu.make_async_copy(
            rows_v.at[b], out_hbm.at[pl.ds(base + j * _CHUNK, _CHUNK)],
            ssems[b])

    for b in range(_NBUF - 1):
        gdesc(b, b).start()

    for b in range(_NBUF):
        gdesc(b, b).wait()
        pb = (b - 1) % _NBUF
        if b > 0:
            sdesc(b - 1, pb).wait()
        gdesc(b + _NBUF - 1, pb).start()
        sdesc(b, b).start()

    @pl.loop(1, _NGROUP - 1)
    def _steady(g):
        j0 = g * _NBUF
        for b in range(_NBUF):
            j = j0 + b
            pb = (b - 1) % _NBUF
            gdesc(j, b).wait()
            sdesc(j - 1, pb).wait()
            gdesc(j + _NBUF - 1, pb).start()
            sdesc(j, b).start()

    j0 = (_NGROUP - 1) * _NBUF
    for b in range(_NBUF):
        j = j0 + b
        pb = (b - 1) % _NBUF
        gdesc(j, b).wait()
        sdesc(j - 1, pb).wait()
        if b == 0:
            gdesc(j + _NBUF - 1, pb).start()
        sdesc(j, b).start()
    sdesc(_NCHUNK - 1, (_NBUF - 1) % _NBUF).wait()


@functools.partial(jax.jit, static_argnames=())
def _emb(idx_flat, weight):
    mesh = plsc.VectorSubcoreMesh(core_axis_name="c", subcore_axis_name="s")
    kern = pl.kernel(
        _emb_body,
        out_type=jax.ShapeDtypeStruct((_NTOK, _DIM), jnp.float32),
        mesh=mesh,
        scratch_types=[
            pltpu.VMEM((_TOK_PER_W,), jnp.int32),
            pltpu.VMEM((_NBUF, _CHUNK, _DIM), jnp.float32),
            pltpu.SemaphoreType.DMA,
            pltpu.SemaphoreType.DMA,
            pltpu.SemaphoreType.DMA,
            pltpu.SemaphoreType.DMA,
            pltpu.SemaphoreType.DMA,
            pltpu.SemaphoreType.DMA,
            pltpu.SemaphoreType.DMA,
            pltpu.SemaphoreType.DMA,
        ],
    )
    return kern(idx_flat, weight)


def kernel(indices, weight):
    idx_flat = indices.reshape(_NTOK).astype(jnp.int32)
    out = _emb(idx_flat, weight)
    return out.reshape(_BATCH, _SEQ, _DIM)
